# Initial kernel scaffold; baseline (speedup 1.0000x reference)
import jax
import jax.numpy as jnp
from jax.experimental import pallas as pl


def kernel(idx, targets, table):
    raise NotImplementedError("write your pallas kernel here")


# speedup vs baseline: 1.0175x; 1.0175x over previous
import functools

import jax
import jax.numpy as jnp
from jax import lax
from jax.experimental import pallas as pl
from jax.experimental.pallas import tpu as pltpu
from jax.experimental.pallas import tpu_sc as plsc

_C = 1000
_NC, _NS = 2, 16
_NW = _NC * _NS
_N = 1024 * 50
_BPW = _N // _NW
_CH = 64
_NCHUNK = _BPW // _CH

def _sc_gather_body(table_hbm, idx_hbm, out_hbm, idx_v, rows_v, sem0, sem1):
    wid = lax.axis_index("s") * _NC + lax.axis_index("c")
    base = wid * _BPW
    pltpu.sync_copy(idx_hbm.at[pl.ds(base, _BPW)], idx_v)
    sems = (sem0, sem1)

    def start(c, buf):
        pltpu.async_copy(
            table_hbm.at[idx_v.at[pl.ds(c * _CH, _CH)]], rows_v.at[buf], sems[buf]
        )

    def step(c, buf):
        @pl.when(c + 1 < _NCHUNK)
        def _():
            start(c + 1, 1 - buf)

        pltpu.make_async_copy(
            table_hbm.at[idx_v.at[pl.ds(c * _CH, _CH)]], rows_v.at[buf], sems[buf]
        ).wait()
        pltpu.sync_copy(rows_v.at[buf], out_hbm.at[pl.ds(base + c * _CH, _CH)])

    start(0, 0)

    def chunk(c, carry):
        @pl.when(lax.rem(c, 2) == 0)
        def _():
            step(c, 0)

        @pl.when(lax.rem(c, 2) == 1)
        def _():
            step(c, 1)

        return carry

    lax.fori_loop(0, _NCHUNK, chunk, 0)


@functools.cache
def _sc_gather():
    mesh = plsc.VectorSubcoreMesh(
        core_axis_name="c", subcore_axis_name="s", num_cores=_NC, num_subcores=_NS
    )
    return pl.kernel(
        _sc_gather_body,
        out_type=jax.ShapeDtypeStruct((_N, _C), jnp.float32),
        mesh=mesh,
        compiler_params=pltpu.CompilerParams(use_tc_tiling_on_sc=False),
        scratch_types=[
            pltpu.VMEM((_BPW,), jnp.int32),
            pltpu.VMEM((2, _CH, _C), jnp.float32),
            pltpu.SemaphoreType.DMA,
            pltpu.SemaphoreType.DMA,
        ],
    )


_BLK = 512
_NBLK = _N // _BLK


def _loss_body(tgt_ref, logits_ref, out_ref, acc_ref):
    i = pl.program_id(0)

    @pl.when(i == 0)
    def _():
        acc_ref[0] = 0.0

    x = logits_ref[...]
    m = jnp.max(x, axis=1, keepdims=True)
    lse = m[:, 0] + jnp.log(jnp.sum(jnp.exp(x - m), axis=1))
    tgt = tgt_ref[0, 0, :]
    cols = lax.broadcasted_iota(jnp.int32, (_BLK, _C), 1)
    picked = jnp.sum(jnp.where(cols == tgt[:, None], x, 0.0), axis=1)
    acc_ref[0] += jnp.sum(lse - picked)

    @pl.when(i == _NBLK - 1)
    def _():
        out_ref[0, 0] = acc_ref[0] / _N


_tc_loss = pl.pallas_call(
    _loss_body,
    grid=(_NBLK,),
    in_specs=[
        pl.BlockSpec((1, 1, _BLK), lambda i: (i, 0, 0)),
        pl.BlockSpec((_BLK, _C), lambda i: (i, 0)),
    ],
    out_specs=pl.BlockSpec((1, 1), lambda i: (0, 0), memory_space=pltpu.SMEM),
    out_shape=jax.ShapeDtypeStruct((1, 1), jnp.float32),
    scratch_shapes=[pltpu.SMEM((1,), jnp.float32)],
)


@jax.jit
def kernel(idx, targets, table):
    flat_idx = idx.reshape(-1).astype(jnp.int32)
    logits_flat = _sc_gather()(table, flat_idx)
    tgt3 = targets.reshape(_NBLK, 1, _BLK).astype(jnp.int32)
    loss = _tc_loss(tgt3, logits_flat)[0, 0]
    return logits_flat.reshape(idx.shape[0], idx.shape[1], _C), loss


# speedup vs baseline: 1.5173x; 1.4911x over previous
import functools

import jax
import jax.numpy as jnp
from jax import lax
from jax.experimental import pallas as pl
from jax.experimental.pallas import tpu as pltpu
from jax.experimental.pallas import tpu_sc as plsc

_C = 1000
_NC, _NS = 2, 16
_NW = _NC * _NS
_N = 1024 * 50
_BPW = _N // _NW
_CH = 40
_NCHUNK = _BPW // _CH
_CP = 1024

def _sc_gather_body(table_hbm, idx_hbm, out_hbm, idx_v, rows_v, sem0, sem1):
    wid = lax.axis_index("s") * _NC + lax.axis_index("c")
    base = wid * _BPW
    pltpu.sync_copy(idx_hbm.at[pl.ds(base, _BPW)], idx_v)
    sems = (sem0, sem1)

    def start(c, buf):
        pltpu.async_copy(
            table_hbm.at[idx_v.at[pl.ds(c * _CH, _CH)]], rows_v.at[buf], sems[buf]
        )

    def step(c, buf):
        @pl.when(c + 1 < _NCHUNK)
        def _():
            start(c + 1, 1 - buf)

        pltpu.make_async_copy(
            table_hbm.at[idx_v.at[pl.ds(c * _CH, _CH)]], rows_v.at[buf], sems[buf]
        ).wait()
        pltpu.sync_copy(rows_v.at[buf], out_hbm.at[pl.ds(base + c * _CH, _CH)])

    start(0, 0)

    def chunk(c, carry):
        @pl.when(lax.rem(c, 2) == 0)
        def _():
            step(c, 0)

        @pl.when(lax.rem(c, 2) == 1)
        def _():
            step(c, 1)

        return carry

    lax.fori_loop(0, _NCHUNK, chunk, 0)


@functools.cache
def _sc_gather():
    mesh = plsc.VectorSubcoreMesh(
        core_axis_name="c", subcore_axis_name="s", num_cores=_NC, num_subcores=_NS
    )
    return pl.kernel(
        _sc_gather_body,
        out_type=jax.ShapeDtypeStruct((_N, _CP), jnp.float32),
        mesh=mesh,
        scratch_types=[
            pltpu.VMEM((_BPW,), jnp.int32),
            pltpu.VMEM((2, _CH, _CP), jnp.float32),
            pltpu.SemaphoreType.DMA,
            pltpu.SemaphoreType.DMA,
        ],
    )


_BLK = 512
_NBLK = _N // _BLK


def _loss_body(tgt_ref, logits_ref, out_ref, loss_ref, acc_ref):
    i = pl.program_id(0)

    @pl.when(i == 0)
    def _():
        acc_ref[0] = 0.0

    x = logits_ref[:, : _C]
    out_ref[...] = x
    m = jnp.max(x, axis=1, keepdims=True)
    lse = m[:, 0] + jnp.log(jnp.sum(jnp.exp(x - m), axis=1))
    tgt = tgt_ref[0, 0, :]
    cols = lax.broadcasted_iota(jnp.int32, (_BLK, _C), 1)
    picked = jnp.sum(jnp.where(cols == tgt[:, None], x, 0.0), axis=1)
    acc_ref[0] += jnp.sum(lse - picked)

    @pl.when(i == _NBLK - 1)
    def _():
        loss_ref[0, 0] = acc_ref[0] / _N


_tc_loss = pl.pallas_call(
    _loss_body,
    grid=(_NBLK,),
    in_specs=[
        pl.BlockSpec((1, 1, _BLK), lambda i: (i, 0, 0)),
        pl.BlockSpec((_BLK, _CP), lambda i: (i, 0)),
    ],
    out_specs=[
        pl.BlockSpec((_BLK, _C), lambda i: (i, 0)),
        pl.BlockSpec((1, 1), lambda i: (0, 0), memory_space=pltpu.SMEM),
    ],
    out_shape=[
        jax.ShapeDtypeStruct((_N, _C), jnp.float32),
        jax.ShapeDtypeStruct((1, 1), jnp.float32),
    ],
    scratch_shapes=[pltpu.SMEM((1,), jnp.float32)],
)


@jax.jit
def kernel(idx, targets, table):
    flat_idx = idx.reshape(-1).astype(jnp.int32)
    table_p = jnp.pad(table, ((0, 0), (0, _CP - _C)))
    logits_pad = _sc_gather()(table_p, flat_idx)
    tgt3 = targets.reshape(_NBLK, 1, _BLK).astype(jnp.int32)
    logits_flat, loss = _tc_loss(tgt3, logits_pad)
    return logits_flat.reshape(idx.shape[0], idx.shape[1], _C), loss[0, 0]


# speedup vs baseline: 2.6232x; 1.7289x over previous
import functools

import jax
import jax.numpy as jnp
from jax import lax
from jax.experimental import pallas as pl
from jax.experimental.pallas import tpu as pltpu
from jax.experimental.pallas import tpu_sc as plsc

_B, _T = 1024, 50
_C = 1000
_NC, _NS = 2, 16
_NW = _NC * _NS
_N = _B * _T
_BPW = _N // _NW
_CH = 40
_NCHUNK = _BPW // _CH
_CP = 1024


def _sc_gather_body(table_hbm, idx_hbm, out_hbm, idx_v, rows_v, sem0, sem1):
    wid = lax.axis_index("s") * _NC + lax.axis_index("c")
    base = wid * _BPW
    pltpu.sync_copy(idx_hbm.at[pl.ds(base, _BPW)], idx_v)
    sems = (sem0, sem1)

    def start(c, buf):
        pltpu.async_copy(
            table_hbm.at[idx_v.at[pl.ds(c * _CH, _CH)]], rows_v.at[buf], sems[buf]
        )

    def step(c, buf):
        @pl.when(c + 1 < _NCHUNK)
        def _():
            start(c + 1, 1 - buf)

        pltpu.make_async_copy(
            table_hbm.at[idx_v.at[pl.ds(c * _CH, _CH)]], rows_v.at[buf], sems[buf]
        ).wait()
        pltpu.sync_copy(rows_v.at[buf], out_hbm.at[pl.ds(base + c * _CH, _CH)])

    start(0, 0)

    def chunk(c, carry):
        @pl.when(lax.rem(c, 2) == 0)
        def _():
            step(c, 0)

        @pl.when(lax.rem(c, 2) == 1)
        def _():
            step(c, 1)

        return carry

    lax.fori_loop(0, _NCHUNK, chunk, 0)


@functools.cache
def _sc_gather():
    mesh = plsc.VectorSubcoreMesh(
        core_axis_name="c", subcore_axis_name="s", num_cores=_NC, num_subcores=_NS
    )
    return pl.kernel(
        _sc_gather_body,
        out_type=jax.ShapeDtypeStruct((_N, _CP), jnp.float32),
        mesh=mesh,
        scratch_types=[
            pltpu.VMEM((_BPW,), jnp.int32),
            pltpu.VMEM((2, _CH, _CP), jnp.float32),
            pltpu.SemaphoreType.DMA,
            pltpu.SemaphoreType.DMA,
        ],
    )


_BLK = 512
_NBLK = _N // _BLK


def _loss_body(tgt_ref, logits_ref, out_ref, loss_ref, acc_ref):
    i = pl.program_id(0)

    @pl.when(i == 0)
    def _():
        acc_ref[0] = 0.0

    x = logits_ref[...]
    valid = lax.broadcasted_iota(jnp.int32, (_BLK, _CP), 1) < _C
    xm = jnp.where(valid, x, -jnp.inf)
    m = jnp.max(xm, axis=1, keepdims=True)
    lse = m[:, 0] + jnp.log(jnp.sum(jnp.exp(xm - m), axis=1))
    tgt = tgt_ref[0, 0, :]
    cols = lax.broadcasted_iota(jnp.int32, (_BLK, _CP), 1)
    picked = jnp.sum(jnp.where(cols == tgt[:, None], x, 0.0), axis=1)
    acc_ref[0] += jnp.sum(lse - picked)

    out_ref[0] = jnp.transpose(x)[: _C, :]

    @pl.when(i == _NBLK - 1)
    def _():
        loss_ref[0, 0] = acc_ref[0] / _N


_tc_loss = pl.pallas_call(
    _loss_body,
    grid=(_NBLK,),
    in_specs=[
        pl.BlockSpec((1, 1, _BLK), lambda i: (i, 0, 0)),
        pl.BlockSpec((_BLK, _CP), lambda i: (i, 0)),
    ],
    out_specs=[
        pl.BlockSpec((1, _C, _BLK), lambda i: (i // 2, 0, i % 2)),
        pl.BlockSpec((1, 1), lambda i: (0, 0), memory_space=pltpu.SMEM),
    ],
    out_shape=[
        jax.ShapeDtypeStruct((_T, _C, _B), jnp.float32),
        jax.ShapeDtypeStruct((1, 1), jnp.float32),
    ],
    scratch_shapes=[pltpu.SMEM((1,), jnp.float32)],
)


@jax.jit
def kernel(idx, targets, table):
    idx_tm = idx.T.reshape(-1).astype(jnp.int32)
    tgt_tm = targets.T.reshape(_NBLK, 1, _BLK).astype(jnp.int32)
    table_p = jnp.pad(table, ((0, 0), (0, _CP - _C)))
    logits_tm = _sc_gather()(table_p, idx_tm)
    logits_tvb, loss = _tc_loss(tgt_tm, logits_tm)
    return jnp.transpose(logits_tvb, (2, 0, 1)), loss[0, 0]


# speedup vs baseline: 2.9164x; 1.1118x over previous
import functools

import jax
import jax.numpy as jnp
from jax import lax
from jax.experimental import pallas as pl
from jax.experimental.pallas import tpu as pltpu
from jax.experimental.pallas import tpu_sc as plsc

_B, _T = 1024, 50
_C = 1000
_NC, _NS = 2, 16
_NW = _NC * _NS
_N = _B * _T
_BPW = _N // _NW
_CH = 40
_NCHUNK = _BPW // _CH
_CP = 1024


def _sc_gather_body(table_hbm, idx_hbm, out_hbm, idx_v, rows_v, sem0, sem1):
    wid = lax.axis_index("s") * _NC + lax.axis_index("c")
    base = wid * _BPW
    pltpu.sync_copy(idx_hbm.at[pl.ds(base, _BPW)], idx_v)
    sems = (sem0, sem1)

    def start(c, buf):
        pltpu.async_copy(
            table_hbm.at[idx_v.at[pl.ds(c * _CH, _CH)]], rows_v.at[buf], sems[buf]
        )

    def step(c, buf):
        @pl.when(c + 1 < _NCHUNK)
        def _():
            start(c + 1, 1 - buf)

        pltpu.make_async_copy(
            table_hbm.at[idx_v.at[pl.ds(c * _CH, _CH)]], rows_v.at[buf], sems[buf]
        ).wait()
        pltpu.sync_copy(rows_v.at[buf], out_hbm.at[pl.ds(base + c * _CH, _CH)])

    start(0, 0)

    def chunk(c, carry):
        @pl.when(lax.rem(c, 2) == 0)
        def _():
            step(c, 0)

        @pl.when(lax.rem(c, 2) == 1)
        def _():
            step(c, 1)

        return carry

    lax.fori_loop(0, _NCHUNK, chunk, 0)


@functools.cache
def _sc_gather():
    mesh = plsc.VectorSubcoreMesh(
        core_axis_name="c", subcore_axis_name="s", num_cores=_NC, num_subcores=_NS
    )
    return pl.kernel(
        _sc_gather_body,
        out_type=jax.ShapeDtypeStruct((_N, _CP), jnp.float32),
        mesh=mesh,
        scratch_types=[
            pltpu.VMEM((_BPW,), jnp.int32),
            pltpu.VMEM((2, _CH, _CP), jnp.float32),
            pltpu.SemaphoreType.DMA,
            pltpu.SemaphoreType.DMA,
        ],
    )


_BLK = 1024
_NBLK = _N // _BLK


def _loss_body(tgt_ref, logits_ref, out_ref, loss_ref, acc_ref):
    i = pl.program_id(0)

    @pl.when(i == 0)
    def _():
        acc_ref[0] = 0.0

    x = logits_ref[...]
    valid = lax.broadcasted_iota(jnp.int32, (_BLK, _CP), 1) < _C
    xm = jnp.where(valid, x, -jnp.inf)
    m = jnp.max(xm, axis=1, keepdims=True)
    lse = m[:, 0] + jnp.log(jnp.sum(jnp.exp(xm - m), axis=1))
    tgt = tgt_ref[0, 0, :]
    cols = lax.broadcasted_iota(jnp.int32, (_BLK, _CP), 1)
    picked = jnp.sum(jnp.where(cols == tgt[:, None], x, 0.0), axis=1)
    acc_ref[0] += jnp.sum(lse - picked)

    out_ref[0] = jnp.transpose(x)[: _C, :]

    @pl.when(i == _NBLK - 1)
    def _():
        loss_ref[0, 0] = acc_ref[0] / _N


_tc_loss = pl.pallas_call(
    _loss_body,
    grid=(_NBLK,),
    in_specs=[
        pl.BlockSpec((1, 1, _BLK), lambda i: (i, 0, 0)),
        pl.BlockSpec((_BLK, _CP), lambda i: (i, 0)),
    ],
    out_specs=[
        pl.BlockSpec((1, _C, _BLK), lambda i: (i, 0, 0)),
        pl.BlockSpec((1, 1), lambda i: (0, 0), memory_space=pltpu.SMEM),
    ],
    out_shape=[
        jax.ShapeDtypeStruct((_T, _C, _B), jnp.float32),
        jax.ShapeDtypeStruct((1, 1), jnp.float32),
    ],
    scratch_shapes=[pltpu.SMEM((1,), jnp.float32)],
)


@jax.jit
def kernel(idx, targets, table):
    idx_tm = idx.T.reshape(-1).astype(jnp.int32)
    tgt_tm = targets.T.reshape(_NBLK, 1, _BLK).astype(jnp.int32)
    table_p = jnp.pad(table, ((0, 0), (0, _CP - _C)))
    logits_tm = _sc_gather()(table_p, idx_tm)
    logits_tvb, loss = _tc_loss(tgt_tm, logits_tm)
    return jnp.transpose(logits_tvb, (2, 0, 1)), loss[0, 0]
